# baseline (speedup 1.0000x reference)
import functools

import jax
import jax.numpy as jnp
from jax import lax
from jax.experimental import pallas as pl
from jax.experimental.pallas import tpu as pltpu
from jax.experimental.pallas import tpu_sc as plsc

N = 10000
E = 320000
D = 128
DE = 16
NC = 2
NS = 16
LANES = 16
NW = NC * NS
EPW = E // NW
CH = 80
NCH = EPW // CH
CPAD = 8
ROWS_DUMP = 640
_SQRT2 = 1.4142135623730951


def _sc_mesh():
    return plsc.VectorSubcoreMesh(core_axis_name="c", subcore_axis_name="s")


_SC_PARAMS = pltpu.CompilerParams(needs_layout_passes=False)


def _radial_call(xpad, src, dst):
    @functools.partial(
        pl.kernel,
        out_type=jax.ShapeDtypeStruct((E,), jnp.float32),
        mesh=_sc_mesh(),
        compiler_params=_SC_PARAMS,
        scratch_types=[
            pltpu.VMEM((N * CPAD,), jnp.float32),
            pltpu.VMEM((EPW,), jnp.int32),
            pltpu.VMEM((EPW,), jnp.int32),
            pltpu.VMEM((RING, CH), jnp.float32),
        ] + [pltpu.SemaphoreType.DMA] * RING,
    )
    def radial_kernel(xpad_hbm, src_hbm, dst_hbm, rad_hbm, ctab, si, di, rv,
                      *sems):
        cid = lax.axis_index("c")
        sid = lax.axis_index("s")
        base = (sid * NC + cid) * EPW
        pltpu.sync_copy(xpad_hbm, ctab)
        pltpu.sync_copy(src_hbm.at[pl.ds(base, EPW)], si)
        pltpu.sync_copy(dst_hbm.at[pl.ds(base, EPW)], di)

        def w_desc(j, bslot):
            return pltpu.make_async_copy(
                rv.at[bslot], rad_hbm.at[pl.ds(base + j * CH, CH)],
                sems[bslot])

        def compute(j, bslot):
            for k in range(CH // LANES):
                sv = si[pl.ds(j * CH + k * LANES, LANES)] * CPAD
                dv = di[pl.ds(j * CH + k * LANES, LANES)] * CPAD
                r = jnp.zeros((LANES,), jnp.float32)
                for c in range(3):
                    xs = plsc.load_gather(ctab, [sv + c])
                    xd = plsc.load_gather(ctab, [dv + c])
                    t = xs - xd
                    r = r + t * t
                rv[bslot, pl.ds(k * LANES, LANES)] = r

        def grp(g, carry):
            for bslot in range(RING):
                j = g * RING + bslot

                @pl.when(g > 0)
                def _wait():
                    w_desc(j - RING, bslot).wait()

                compute(j, bslot)
                w_desc(j, bslot).start()
            return carry

        lax.fori_loop(0, NGRP, grp, 0)
        for bslot in range(RING):
            w_desc((NGRP - 1) * RING + bslot, bslot).wait()

    return radial_kernel(xpad, src, dst)


RING = 5
NGRP = NCH // RING
SCH = 40
SNCH = EPW // SCH
SNGRP = SNCH // RING


def _gather_call(a, b, src, dst):
    @functools.partial(
        pl.kernel,
        out_type=jax.ShapeDtypeStruct((E, D), jnp.float32),
        mesh=_sc_mesh(),
        compiler_params=_SC_PARAMS,
        scratch_types=[
            pltpu.VMEM((EPW,), jnp.int32),
            pltpu.VMEM((EPW,), jnp.int32),
            pltpu.VMEM((RING, CH, D), jnp.float32),
            pltpu.VMEM((RING, CH, D), jnp.float32),
        ] + [pltpu.SemaphoreType.DMA] * (3 * RING),
    )
    def gather_kernel(a_hbm, b_hbm, src_hbm, dst_hbm, g_hbm,
                      si, di, b1, b2, *sems):
        sg1 = sems[0:RING]
        sg2 = sems[RING:2 * RING]
        sw = sems[2 * RING:3 * RING]
        cid = lax.axis_index("c")
        sid = lax.axis_index("s")
        wid = sid * NC + cid
        base = wid * EPW
        pltpu.sync_copy(src_hbm.at[pl.ds(base, EPW)], si)
        pltpu.sync_copy(dst_hbm.at[pl.ds(base, EPW)], di)

        def g_desc(j, bslot):
            return (pltpu.make_async_copy(a_hbm.at[si.at[pl.ds(j * CH, CH)]],
                                          b1.at[bslot], sg1[bslot]),
                    pltpu.make_async_copy(b_hbm.at[di.at[pl.ds(j * CH, CH)]],
                                          b2.at[bslot], sg2[bslot]))

        def w_desc(j, bslot):
            off = base + j * CH
            return pltpu.make_async_copy(b1.at[bslot],
                                         g_hbm.at[pl.ds(off, CH)], sw[bslot])

        def add_rows(bslot):
            def row(r, carry):
                for c in range(D // LANES):
                    sl = pl.ds(c * LANES, LANES)
                    b1[bslot, r, sl] = b1[bslot, r, sl] + b2[bslot, r, sl]
                return carry
            lax.fori_loop(0, CH, row, 0)

        for bslot in range(RING):
            for d in g_desc(bslot, bslot):
                d.start()

        def grp(g, carry):
            for bslot in range(RING):
                j = g * RING + bslot
                for d in g_desc(j, bslot):
                    d.wait()
                add_rows(bslot)
                w_desc(j, bslot).start()

            @pl.when(g < NGRP - 1)
            def _prefetch():
                for bslot in range(RING):
                    j = g * RING + bslot
                    w_desc(j, bslot).wait()
                    for d in g_desc(j + RING, bslot):
                        d.start()

            return carry

        lax.fori_loop(0, NGRP, grp, 0)
        for bslot in range(RING):
            w_desc((NGRP - 1) * RING + bslot, bslot).wait()

    return gather_kernel(a, b, src, dst)


def _scatter_call(msg, dst, zeros):
    @functools.partial(
        pl.kernel,
        out_type=jax.ShapeDtypeStruct((NC, N, D), jnp.float32),
        mesh=_sc_mesh(),
        compiler_params=_SC_PARAMS,
        scratch_types=[
            pltpu.VMEM((EPW,), jnp.int32),
            pltpu.VMEM((RING, SCH, D), jnp.float32),
            pltpu.VMEM_SHARED((N, D), jnp.float32),
        ] + [pltpu.SemaphoreType.DMA] * (2 * RING),
    )
    def scatter_kernel(msg_hbm, dst_hbm, zero_hbm, out_hbm, di, buf, acc,
                       *sems):
        sl = sems[0:RING]
        ss = sems[RING:2 * RING]
        cid = lax.axis_index("c")
        sid = lax.axis_index("s")
        wid = sid * NC + cid
        base = wid * EPW

        @pl.when(sid < NS - 1)
        def _init():
            pltpu.sync_copy(zero_hbm.at[pl.ds(sid * ROWS_DUMP, ROWS_DUMP)],
                            acc.at[pl.ds(sid * ROWS_DUMP, ROWS_DUMP)])

        @pl.when(sid == NS - 1)
        def _init_last():
            rem = N - (NS - 1) * ROWS_DUMP
            pltpu.sync_copy(zero_hbm.at[pl.ds((NS - 1) * ROWS_DUMP, rem)],
                            acc.at[pl.ds((NS - 1) * ROWS_DUMP, rem)])

        pltpu.sync_copy(dst_hbm.at[pl.ds(base, EPW)], di)
        plsc.subcore_barrier()

        def l_desc(j, bslot):
            off = base + j * SCH
            return pltpu.make_async_copy(msg_hbm.at[pl.ds(off, SCH)],
                                         buf.at[bslot], sl[bslot])

        def s_desc(j, bslot):
            return pltpu.make_async_copy(
                buf.at[bslot], acc.at[di.at[pl.ds(j * SCH, SCH)]], ss[bslot])

        for bslot in range(RING):
            l_desc(bslot, bslot).start()

        def grp(g, carry):
            for bslot in range(RING):
                j = g * RING + bslot
                l_desc(j, bslot).wait()
                s_desc(j, bslot).start(add=True)

            @pl.when(g < SNGRP - 1)
            def _prefetch():
                for bslot in range(RING):
                    j = g * RING + bslot
                    s_desc(j, bslot).wait()
                    l_desc(j + RING, bslot).start()

            return carry

        lax.fori_loop(0, SNGRP, grp, 0)
        for bslot in range(RING):
            s_desc((SNGRP - 1) * RING + bslot, bslot).wait()
        plsc.subcore_barrier()

        @pl.when(sid < NS - 1)
        def _dump():
            pltpu.sync_copy(
                acc.at[pl.ds(sid * ROWS_DUMP, ROWS_DUMP)],
                out_hbm.at[cid, pl.ds(sid * ROWS_DUMP, ROWS_DUMP)])

        @pl.when(sid == NS - 1)
        def _dump_last():
            rem = N - (NS - 1) * ROWS_DUMP
            pltpu.sync_copy(
                acc.at[pl.ds((NS - 1) * ROWS_DUMP, rem)],
                out_hbm.at[cid, pl.ds((NS - 1) * ROWS_DUMP, rem)])

    return scatter_kernel(msg, dst, zeros)


def _proj_body(h_ref, wa_ref, wb_ref, ba_ref, a_ref, b_ref):
    h = h_ref[...]
    a_ref[...] = jnp.dot(h, wa_ref[...],
                         preferred_element_type=jnp.float32) + ba_ref[...]
    b_ref[...] = jnp.dot(h, wb_ref[...], preferred_element_type=jnp.float32)


def _proj_call(h, wa, wb, eb1):
    RB = 1000
    return pl.pallas_call(
        _proj_body,
        grid=(N // RB,),
        in_specs=[
            pl.BlockSpec((RB, D), lambda i: (i, 0)),
            pl.BlockSpec((D, D), lambda i: (0, 0)),
            pl.BlockSpec((D, D), lambda i: (0, 0)),
            pl.BlockSpec((1, D), lambda i: (0, 0)),
        ],
        out_specs=[
            pl.BlockSpec((RB, D), lambda i: (i, 0)),
            pl.BlockSpec((RB, D), lambda i: (i, 0)),
        ],
        out_shape=[jax.ShapeDtypeStruct((N, D), jnp.float32)] * 2,
    )(h, wa, wb, eb1.reshape(1, D))


EB = 2560
EBR = EB // 128


def _edge_body(g_ref, rad_ref, ea_ref, wr_ref, we_ref, w2_ref, b2_ref,
               msg_ref):
    eaC = lax.dot_general(ea_ref[...].astype(jnp.bfloat16),
                          we_ref[...].astype(jnp.bfloat16),
                          (((0,), (0,)), ((), ())),
                          preferred_element_type=jnp.float32)
    t3 = (g_ref[...].reshape(EBR, 128, D) + eaC.reshape(EBR, 128, D)
          + rad_ref[0][:, :, None] * wr_ref[...][None, :, :]
          ).astype(jnp.bfloat16)
    m = t3 * jax.nn.sigmoid(t3)
    u = (jnp.dot(m.reshape(EB, D), w2_ref[...].astype(jnp.bfloat16),
                 preferred_element_type=jnp.float32)
         + b2_ref[...]).astype(jnp.bfloat16)
    msg_ref[...] = (u * jax.nn.sigmoid(u)).astype(jnp.float32)


def _edge_call(g, rad2, eaT, wr, we, w2, b2):
    return pl.pallas_call(
        _edge_body,
        grid=(E // EB,),
        in_specs=[
            pl.BlockSpec((EB, D), lambda i: (i, 0)),
            pl.BlockSpec((1, EBR, 128), lambda i: (i, 0, 0)),
            pl.BlockSpec((DE, EB), lambda i: (0, i)),
            pl.BlockSpec((1, D), lambda i: (0, 0)),
            pl.BlockSpec((DE, D), lambda i: (0, 0)),
            pl.BlockSpec((D, D), lambda i: (0, 0)),
            pl.BlockSpec((1, D), lambda i: (0, 0)),
        ],
        out_specs=pl.BlockSpec((EB, D), lambda i: (i, 0)),
        out_shape=jax.ShapeDtypeStruct((E, D), jnp.float32),
    )(g, rad2, eaT, wr.reshape(1, D), we, w2, b2.reshape(1, D))


def _node_body(h_ref, p_ref, w1a_ref, w1b_ref, b1_ref, w2_ref, b2_ref,
               g_ref, be_ref, o_ref):
    h = h_ref[...]
    hn = p_ref[0] + p_ref[1]
    t = (jnp.dot(h, w1a_ref[...], preferred_element_type=jnp.float32)
         + jnp.dot(hn, w1b_ref[...], preferred_element_type=jnp.float32)
         + b1_ref[...])
    u = t * jax.nn.sigmoid(t)
    v = jnp.dot(u, w2_ref[...], preferred_element_type=jnp.float32) + b2_ref[...]
    mu = jnp.mean(v, axis=1, keepdims=True)
    dlt = v - mu
    var = jnp.mean(dlt * dlt, axis=1, keepdims=True)
    vn = dlt * lax.rsqrt(var + 1e-5) * g_ref[...] + be_ref[...]
    gl = 0.5 * vn * (1.0 + lax.erf(vn / _SQRT2))
    o_ref[...] = gl + h


def _node_proj_body(h_ref, p_ref, w1a_ref, w1b_ref, b1_ref, w2_ref, b2_ref,
                    g_ref, be_ref, nwa_ref, nwb_ref, nba_ref,
                    o_ref, a_ref, b_ref):
    h = h_ref[...]
    hn = p_ref[0] + p_ref[1]
    t = (jnp.dot(h, w1a_ref[...], preferred_element_type=jnp.float32)
         + jnp.dot(hn, w1b_ref[...], preferred_element_type=jnp.float32)
         + b1_ref[...])
    u = t * jax.nn.sigmoid(t)
    v = jnp.dot(u, w2_ref[...], preferred_element_type=jnp.float32) + b2_ref[...]
    mu = jnp.mean(v, axis=1, keepdims=True)
    dlt = v - mu
    var = jnp.mean(dlt * dlt, axis=1, keepdims=True)
    vn = dlt * lax.rsqrt(var + 1e-5) * g_ref[...] + be_ref[...]
    gl = 0.5 * vn * (1.0 + lax.erf(vn / _SQRT2))
    ho = gl + h
    o_ref[...] = ho
    a_ref[...] = jnp.dot(ho, nwa_ref[...],
                         preferred_element_type=jnp.float32) + nba_ref[...]
    b_ref[...] = jnp.dot(ho, nwb_ref[...], preferred_element_type=jnp.float32)


def _node_proj_call(h, parts, w1a, w1b, b1, w2, b2, g, be, nwa, nwb, nba):
    RB = 1000
    wspec = pl.BlockSpec((D, D), lambda i: (0, 0))
    vspec = pl.BlockSpec((1, D), lambda i: (0, 0))
    rspec = pl.BlockSpec((RB, D), lambda i: (i, 0))
    return pl.pallas_call(
        _node_proj_body,
        grid=(N // RB,),
        in_specs=[
            rspec,
            pl.BlockSpec((NC, RB, D), lambda i: (0, i, 0)),
            wspec, wspec, vspec, wspec, vspec, vspec, vspec,
            wspec, wspec, vspec,
        ],
        out_specs=[rspec, rspec, rspec],
        out_shape=[jax.ShapeDtypeStruct((N, D), jnp.float32)] * 3,
    )(h, parts, w1a, w1b, b1.reshape(1, D), w2, b2.reshape(1, D),
      g.reshape(1, D), be.reshape(1, D), nwa, nwb, nba.reshape(1, D))


def _node_call(h, parts, w1a, w1b, b1, w2, b2, g, be):
    RB = 1000
    return pl.pallas_call(
        _node_body,
        grid=(N // RB,),
        in_specs=[
            pl.BlockSpec((RB, D), lambda i: (i, 0)),
            pl.BlockSpec((NC, RB, D), lambda i: (0, i, 0)),
            pl.BlockSpec((D, D), lambda i: (0, 0)),
            pl.BlockSpec((D, D), lambda i: (0, 0)),
            pl.BlockSpec((1, D), lambda i: (0, 0)),
            pl.BlockSpec((D, D), lambda i: (0, 0)),
            pl.BlockSpec((1, D), lambda i: (0, 0)),
            pl.BlockSpec((1, D), lambda i: (0, 0)),
            pl.BlockSpec((1, D), lambda i: (0, 0)),
        ],
        out_specs=pl.BlockSpec((RB, D), lambda i: (i, 0)),
        out_shape=jax.ShapeDtypeStruct((N, D), jnp.float32),
    )(h, parts, w1a, w1b, b1.reshape(1, D), w2, b2.reshape(1, D),
      g.reshape(1, D), be.reshape(1, D))


def kernel(node_features, coord_features, edge_features, edge_index, params):
    src = edge_index[0]
    dst = edge_index[1]
    xpad = jnp.pad(coord_features, ((0, 0), (0, CPAD - 3))).reshape(-1)
    rad2 = _radial_call(xpad, src, dst).reshape(E // EB, EB // 128, 128)
    eaT = edge_features.T
    zeros = jnp.zeros((N, D), jnp.float32)

    p0, p1 = params[0], params[1]

    h = node_features
    A, B = _proj_call(h, p0['eW1'][:D], p0['eW1'][D:2 * D], p0['eb1'])
    g = _gather_call(A, B, src, dst)
    msg = _edge_call(g, rad2, eaT, p0['eW1'][2 * D],
                     p0['eW1'][2 * D + 1:], p0['eW2'], p0['eb2'])
    parts = _scatter_call(msg, dst, zeros)
    h, A, B = _node_proj_call(h, parts, p0['nW1'][:D], p0['nW1'][D:],
                              p0['nb1'], p0['nW2'], p0['nb2'], p0['ln_g'],
                              p0['ln_b'], p1['eW1'][:D], p1['eW1'][D:2 * D],
                              p1['eb1'])
    g = _gather_call(A, B, src, dst)
    msg = _edge_call(g, rad2, eaT, p1['eW1'][2 * D],
                     p1['eW1'][2 * D + 1:], p1['eW2'], p1['eb2'])
    parts = _scatter_call(msg, dst, zeros)
    h = _node_call(h, parts, p1['nW1'][:D], p1['nW1'][D:], p1['nb1'],
                   p1['nW2'], p1['nb2'], p1['ln_g'], p1['ln_b'])
    return h


# speedup vs baseline: 1.0922x; 1.0922x over previous
import functools

import jax
import jax.numpy as jnp
from jax import lax
from jax.experimental import pallas as pl
from jax.experimental.pallas import tpu as pltpu
from jax.experimental.pallas import tpu_sc as plsc

N = 10000
E = 320000
D = 128
DE = 16
NC = 2
NS = 16
LANES = 16
NW = NC * NS
EPW = E // NW
CH = 80
NCH = EPW // CH
CPAD = 8
ROWS_DUMP = 640
_SQRT2 = 1.4142135623730951


def _sc_mesh():
    return plsc.VectorSubcoreMesh(core_axis_name="c", subcore_axis_name="s")


_SC_PARAMS = pltpu.CompilerParams(needs_layout_passes=False)


def _radial_call(xpad, src, dst):
    @functools.partial(
        pl.kernel,
        out_type=jax.ShapeDtypeStruct((E,), jnp.float32),
        mesh=_sc_mesh(),
        compiler_params=_SC_PARAMS,
        scratch_types=[
            pltpu.VMEM((N * CPAD,), jnp.float32),
            pltpu.VMEM((EPW,), jnp.int32),
            pltpu.VMEM((EPW,), jnp.int32),
            pltpu.VMEM((RING, CH), jnp.float32),
        ] + [pltpu.SemaphoreType.DMA] * RING,
    )
    def radial_kernel(xpad_hbm, src_hbm, dst_hbm, rad_hbm, ctab, si, di, rv,
                      *sems):
        cid = lax.axis_index("c")
        sid = lax.axis_index("s")
        base = (sid * NC + cid) * EPW
        pltpu.sync_copy(xpad_hbm, ctab)
        pltpu.sync_copy(src_hbm.at[pl.ds(base, EPW)], si)
        pltpu.sync_copy(dst_hbm.at[pl.ds(base, EPW)], di)

        def w_desc(j, bslot):
            return pltpu.make_async_copy(
                rv.at[bslot], rad_hbm.at[pl.ds(base + j * CH, CH)],
                sems[bslot])

        def compute(j, bslot):
            for k in range(CH // LANES):
                sv = si[pl.ds(j * CH + k * LANES, LANES)] * CPAD
                dv = di[pl.ds(j * CH + k * LANES, LANES)] * CPAD
                r = jnp.zeros((LANES,), jnp.float32)
                for c in range(3):
                    xs = plsc.load_gather(ctab, [sv + c])
                    xd = plsc.load_gather(ctab, [dv + c])
                    t = xs - xd
                    r = r + t * t
                rv[bslot, pl.ds(k * LANES, LANES)] = r

        def grp(g, carry):
            for bslot in range(RING):
                j = g * RING + bslot

                @pl.when(g > 0)
                def _wait():
                    w_desc(j - RING, bslot).wait()

                compute(j, bslot)
                w_desc(j, bslot).start()
            return carry

        lax.fori_loop(0, NGRP, grp, 0)
        for bslot in range(RING):
            w_desc((NGRP - 1) * RING + bslot, bslot).wait()

    return radial_kernel(xpad, src, dst)


RING = 5
NGRP = NCH // RING
SCH = 40
SNCH = EPW // SCH
SNGRP = SNCH // RING


def _gather_call(a, b, src, dst):
    @functools.partial(
        pl.kernel,
        out_type=jax.ShapeDtypeStruct((E, D), jnp.float32),
        mesh=_sc_mesh(),
        compiler_params=_SC_PARAMS,
        scratch_types=[
            pltpu.VMEM((EPW,), jnp.int32),
            pltpu.VMEM((EPW,), jnp.int32),
            pltpu.VMEM((RING, CH, D), jnp.float32),
            pltpu.VMEM((RING, CH, D), jnp.float32),
        ] + [pltpu.SemaphoreType.DMA] * (3 * RING),
    )
    def gather_kernel(a_hbm, b_hbm, src_hbm, dst_hbm, g_hbm,
                      si, di, b1, b2, *sems):
        sg1 = sems[0:RING]
        sg2 = sems[RING:2 * RING]
        sw = sems[2 * RING:3 * RING]
        cid = lax.axis_index("c")
        sid = lax.axis_index("s")
        wid = sid * NC + cid
        base = wid * EPW
        pltpu.sync_copy(src_hbm.at[pl.ds(base, EPW)], si)
        pltpu.sync_copy(dst_hbm.at[pl.ds(base, EPW)], di)

        def g_desc(j, bslot):
            return (pltpu.make_async_copy(a_hbm.at[si.at[pl.ds(j * CH, CH)]],
                                          b1.at[bslot], sg1[bslot]),
                    pltpu.make_async_copy(b_hbm.at[di.at[pl.ds(j * CH, CH)]],
                                          b2.at[bslot], sg2[bslot]))

        def w_desc(j, bslot):
            off = base + j * CH
            return pltpu.make_async_copy(b1.at[bslot],
                                         g_hbm.at[pl.ds(off, CH)], sw[bslot])

        def add_rows(bslot):
            def row(r, carry):
                for c in range(D // LANES):
                    sl = pl.ds(c * LANES, LANES)
                    b1[bslot, r, sl] = b1[bslot, r, sl] + b2[bslot, r, sl]
                return carry
            lax.fori_loop(0, CH, row, 0)

        for bslot in range(RING):
            for d in g_desc(bslot, bslot):
                d.start()

        def grp(g, carry):
            for bslot in range(RING):
                j = g * RING + bslot
                for d in g_desc(j, bslot):
                    d.wait()
                add_rows(bslot)
                w_desc(j, bslot).start()

            @pl.when(g < NGRP - 1)
            def _prefetch():
                for bslot in range(RING):
                    j = g * RING + bslot
                    w_desc(j, bslot).wait()
                    for d in g_desc(j + RING, bslot):
                        d.start()

            return carry

        lax.fori_loop(0, NGRP, grp, 0)
        for bslot in range(RING):
            w_desc((NGRP - 1) * RING + bslot, bslot).wait()

    return gather_kernel(a, b, src, dst)


def _scatter_call(msg, dst, zeros):
    @functools.partial(
        pl.kernel,
        out_type=jax.ShapeDtypeStruct((NC, N, D), jnp.float32),
        mesh=_sc_mesh(),
        compiler_params=_SC_PARAMS,
        scratch_types=[
            pltpu.VMEM((EPW,), jnp.int32),
            pltpu.VMEM((RING, SCH, D), jnp.float32),
            pltpu.VMEM_SHARED((N, D), jnp.float32),
        ] + [pltpu.SemaphoreType.DMA] * (2 * RING),
    )
    def scatter_kernel(msg_hbm, dst_hbm, zero_hbm, out_hbm, di, buf, acc,
                       *sems):
        sl = sems[0:RING]
        ss = sems[RING:2 * RING]
        cid = lax.axis_index("c")
        sid = lax.axis_index("s")
        wid = sid * NC + cid
        base = wid * EPW

        @pl.when(sid < NS - 1)
        def _init():
            pltpu.sync_copy(zero_hbm.at[pl.ds(sid * ROWS_DUMP, ROWS_DUMP)],
                            acc.at[pl.ds(sid * ROWS_DUMP, ROWS_DUMP)])

        @pl.when(sid == NS - 1)
        def _init_last():
            rem = N - (NS - 1) * ROWS_DUMP
            pltpu.sync_copy(zero_hbm.at[pl.ds((NS - 1) * ROWS_DUMP, rem)],
                            acc.at[pl.ds((NS - 1) * ROWS_DUMP, rem)])

        pltpu.sync_copy(dst_hbm.at[pl.ds(base, EPW)], di)
        plsc.subcore_barrier()

        def l_desc(j, bslot):
            off = base + j * SCH
            return pltpu.make_async_copy(msg_hbm.at[pl.ds(off, SCH)],
                                         buf.at[bslot], sl[bslot])

        def s_desc(j, bslot):
            return pltpu.make_async_copy(
                buf.at[bslot], acc.at[di.at[pl.ds(j * SCH, SCH)]], ss[bslot])

        for bslot in range(RING):
            l_desc(bslot, bslot).start()

        def grp(g, carry):
            for bslot in range(RING):
                j = g * RING + bslot
                l_desc(j, bslot).wait()
                s_desc(j, bslot).start(add=True)

            @pl.when(g < SNGRP - 1)
            def _prefetch():
                for bslot in range(RING):
                    j = g * RING + bslot
                    s_desc(j, bslot).wait()
                    l_desc(j + RING, bslot).start()

            return carry

        lax.fori_loop(0, SNGRP, grp, 0)
        for bslot in range(RING):
            s_desc((SNGRP - 1) * RING + bslot, bslot).wait()
        plsc.subcore_barrier()

        @pl.when(sid < NS - 1)
        def _dump():
            pltpu.sync_copy(
                acc.at[pl.ds(sid * ROWS_DUMP, ROWS_DUMP)],
                out_hbm.at[cid, pl.ds(sid * ROWS_DUMP, ROWS_DUMP)])

        @pl.when(sid == NS - 1)
        def _dump_last():
            rem = N - (NS - 1) * ROWS_DUMP
            pltpu.sync_copy(
                acc.at[pl.ds((NS - 1) * ROWS_DUMP, rem)],
                out_hbm.at[cid, pl.ds((NS - 1) * ROWS_DUMP, rem)])

    return scatter_kernel(msg, dst, zeros)


def _proj_body(h_ref, wa_ref, wb_ref, ba_ref, a_ref, b_ref):
    h = h_ref[...]
    a_ref[...] = jnp.dot(h, wa_ref[...],
                         preferred_element_type=jnp.float32) + ba_ref[...]
    b_ref[...] = jnp.dot(h, wb_ref[...], preferred_element_type=jnp.float32)


def _proj_call(h, wa, wb, eb1):
    RB = 1000
    return pl.pallas_call(
        _proj_body,
        grid=(N // RB,),
        in_specs=[
            pl.BlockSpec((RB, D), lambda i: (i, 0)),
            pl.BlockSpec((D, D), lambda i: (0, 0)),
            pl.BlockSpec((D, D), lambda i: (0, 0)),
            pl.BlockSpec((1, D), lambda i: (0, 0)),
        ],
        out_specs=[
            pl.BlockSpec((RB, D), lambda i: (i, 0)),
            pl.BlockSpec((RB, D), lambda i: (i, 0)),
        ],
        out_shape=[jax.ShapeDtypeStruct((N, D), jnp.float32)] * 2,
    )(h, wa, wb, eb1.reshape(1, D))


EB = 6400
EBR = EB // 128


def _edge_body(g_ref, rad_ref, ea_ref, wr_ref, we_ref, w2_ref, b2_ref,
               msg_ref):
    eaC = lax.dot_general(ea_ref[...].astype(jnp.bfloat16),
                          we_ref[...].astype(jnp.bfloat16),
                          (((0,), (0,)), ((), ())),
                          preferred_element_type=jnp.float32)
    t3 = (g_ref[...].reshape(EBR, 128, D) + eaC.reshape(EBR, 128, D)
          + rad_ref[0][:, :, None] * wr_ref[...][None, :, :])
    m = t3 * jax.nn.sigmoid(t3)
    u = (jnp.dot(m.reshape(EB, D).astype(jnp.bfloat16),
                 w2_ref[...].astype(jnp.bfloat16),
                 preferred_element_type=jnp.float32) + b2_ref[...])
    msg_ref[...] = u * jax.nn.sigmoid(u)


def _edge_call(g, rad2, eaT, wr, we, w2, b2):
    return pl.pallas_call(
        _edge_body,
        grid=(E // EB,),
        in_specs=[
            pl.BlockSpec((EB, D), lambda i: (i, 0)),
            pl.BlockSpec((1, EBR, 128), lambda i: (i, 0, 0)),
            pl.BlockSpec((DE, EB), lambda i: (0, i)),
            pl.BlockSpec((1, D), lambda i: (0, 0)),
            pl.BlockSpec((DE, D), lambda i: (0, 0)),
            pl.BlockSpec((D, D), lambda i: (0, 0)),
            pl.BlockSpec((1, D), lambda i: (0, 0)),
        ],
        out_specs=pl.BlockSpec((EB, D), lambda i: (i, 0)),
        out_shape=jax.ShapeDtypeStruct((E, D), jnp.float32),
    )(g, rad2, eaT, wr.reshape(1, D), we, w2, b2.reshape(1, D))


def _node_body(h_ref, p_ref, w1a_ref, w1b_ref, b1_ref, w2_ref, b2_ref,
               g_ref, be_ref, o_ref):
    h = h_ref[...]
    hn = p_ref[0] + p_ref[1]
    t = (jnp.dot(h, w1a_ref[...], preferred_element_type=jnp.float32)
         + jnp.dot(hn, w1b_ref[...], preferred_element_type=jnp.float32)
         + b1_ref[...])
    u = t * jax.nn.sigmoid(t)
    v = jnp.dot(u, w2_ref[...], preferred_element_type=jnp.float32) + b2_ref[...]
    mu = jnp.mean(v, axis=1, keepdims=True)
    dlt = v - mu
    var = jnp.mean(dlt * dlt, axis=1, keepdims=True)
    vn = dlt * lax.rsqrt(var + 1e-5) * g_ref[...] + be_ref[...]
    gl = 0.5 * vn * (1.0 + lax.erf(vn / _SQRT2))
    o_ref[...] = gl + h


def _node_proj_body(h_ref, p_ref, w1a_ref, w1b_ref, b1_ref, w2_ref, b2_ref,
                    g_ref, be_ref, nwa_ref, nwb_ref, nba_ref,
                    o_ref, a_ref, b_ref):
    h = h_ref[...]
    hn = p_ref[0] + p_ref[1]
    t = (jnp.dot(h, w1a_ref[...], preferred_element_type=jnp.float32)
         + jnp.dot(hn, w1b_ref[...], preferred_element_type=jnp.float32)
         + b1_ref[...])
    u = t * jax.nn.sigmoid(t)
    v = jnp.dot(u, w2_ref[...], preferred_element_type=jnp.float32) + b2_ref[...]
    mu = jnp.mean(v, axis=1, keepdims=True)
    dlt = v - mu
    var = jnp.mean(dlt * dlt, axis=1, keepdims=True)
    vn = dlt * lax.rsqrt(var + 1e-5) * g_ref[...] + be_ref[...]
    gl = 0.5 * vn * (1.0 + lax.erf(vn / _SQRT2))
    ho = gl + h
    o_ref[...] = ho
    a_ref[...] = jnp.dot(ho, nwa_ref[...],
                         preferred_element_type=jnp.float32) + nba_ref[...]
    b_ref[...] = jnp.dot(ho, nwb_ref[...], preferred_element_type=jnp.float32)


def _node_proj_call(h, parts, w1a, w1b, b1, w2, b2, g, be, nwa, nwb, nba):
    RB = 1000
    wspec = pl.BlockSpec((D, D), lambda i: (0, 0))
    vspec = pl.BlockSpec((1, D), lambda i: (0, 0))
    rspec = pl.BlockSpec((RB, D), lambda i: (i, 0))
    return pl.pallas_call(
        _node_proj_body,
        grid=(N // RB,),
        in_specs=[
            rspec,
            pl.BlockSpec((NC, RB, D), lambda i: (0, i, 0)),
            wspec, wspec, vspec, wspec, vspec, vspec, vspec,
            wspec, wspec, vspec,
        ],
        out_specs=[rspec, rspec, rspec],
        out_shape=[jax.ShapeDtypeStruct((N, D), jnp.float32)] * 3,
    )(h, parts, w1a, w1b, b1.reshape(1, D), w2, b2.reshape(1, D),
      g.reshape(1, D), be.reshape(1, D), nwa, nwb, nba.reshape(1, D))


def _node_call(h, parts, w1a, w1b, b1, w2, b2, g, be):
    RB = 1000
    return pl.pallas_call(
        _node_body,
        grid=(N // RB,),
        in_specs=[
            pl.BlockSpec((RB, D), lambda i: (i, 0)),
            pl.BlockSpec((NC, RB, D), lambda i: (0, i, 0)),
            pl.BlockSpec((D, D), lambda i: (0, 0)),
            pl.BlockSpec((D, D), lambda i: (0, 0)),
            pl.BlockSpec((1, D), lambda i: (0, 0)),
            pl.BlockSpec((D, D), lambda i: (0, 0)),
            pl.BlockSpec((1, D), lambda i: (0, 0)),
            pl.BlockSpec((1, D), lambda i: (0, 0)),
            pl.BlockSpec((1, D), lambda i: (0, 0)),
        ],
        out_specs=pl.BlockSpec((RB, D), lambda i: (i, 0)),
        out_shape=jax.ShapeDtypeStruct((N, D), jnp.float32),
    )(h, parts, w1a, w1b, b1.reshape(1, D), w2, b2.reshape(1, D),
      g.reshape(1, D), be.reshape(1, D))


def kernel(node_features, coord_features, edge_features, edge_index, params):
    src = edge_index[0]
    dst = edge_index[1]
    xpad = jnp.pad(coord_features, ((0, 0), (0, CPAD - 3))).reshape(-1)
    rad2 = _radial_call(xpad, src, dst).reshape(E // EB, EB // 128, 128)
    eaT = edge_features.T
    zeros = jnp.zeros((N, D), jnp.float32)

    p0, p1 = params[0], params[1]

    h = node_features
    A, B = _proj_call(h, p0['eW1'][:D], p0['eW1'][D:2 * D], p0['eb1'])
    g = _gather_call(A, B, src, dst)
    msg = _edge_call(g, rad2, eaT, p0['eW1'][2 * D],
                     p0['eW1'][2 * D + 1:], p0['eW2'], p0['eb2'])
    parts = _scatter_call(msg, dst, zeros)
    h, A, B = _node_proj_call(h, parts, p0['nW1'][:D], p0['nW1'][D:],
                              p0['nb1'], p0['nW2'], p0['nb2'], p0['ln_g'],
                              p0['ln_b'], p1['eW1'][:D], p1['eW1'][D:2 * D],
                              p1['eb1'])
    g = _gather_call(A, B, src, dst)
    msg = _edge_call(g, rad2, eaT, p1['eW1'][2 * D],
                     p1['eW1'][2 * D + 1:], p1['eW2'], p1['eb2'])
    parts = _scatter_call(msg, dst, zeros)
    h = _node_call(h, parts, p1['nW1'][:D], p1['nW1'][D:], p1['nb1'],
                   p1['nW2'], p1['nb2'], p1['ln_g'], p1['ln_b'])
    return h


# speedup vs baseline: 1.1264x; 1.0313x over previous
import functools

import jax
import jax.numpy as jnp
from jax import lax
from jax.experimental import pallas as pl
from jax.experimental.pallas import tpu as pltpu
from jax.experimental.pallas import tpu_sc as plsc

N = 10000
E = 320000
D = 128
DE = 16
NC = 2
NS = 16
LANES = 16
NW = NC * NS
EPW = E // NW
CH = 80
NCH = EPW // CH
CPAD = 8
ROWS_DUMP = 640
_SQRT2 = 1.4142135623730951


def _sc_mesh():
    return plsc.VectorSubcoreMesh(core_axis_name="c", subcore_axis_name="s")


_SC_PARAMS = pltpu.CompilerParams(needs_layout_passes=False)


def _radial_call(xpad, src, dst):
    @functools.partial(
        pl.kernel,
        out_type=jax.ShapeDtypeStruct((E,), jnp.float32),
        mesh=_sc_mesh(),
        compiler_params=_SC_PARAMS,
        scratch_types=[
            pltpu.VMEM((N * CPAD,), jnp.float32),
            pltpu.VMEM((EPW,), jnp.int32),
            pltpu.VMEM((EPW,), jnp.int32),
            pltpu.VMEM((RING, CH), jnp.float32),
        ] + [pltpu.SemaphoreType.DMA] * RING,
    )
    def radial_kernel(xpad_hbm, src_hbm, dst_hbm, rad_hbm, ctab, si, di, rv,
                      *sems):
        cid = lax.axis_index("c")
        sid = lax.axis_index("s")
        base = (sid * NC + cid) * EPW
        pltpu.sync_copy(xpad_hbm, ctab)
        pltpu.sync_copy(src_hbm.at[pl.ds(base, EPW)], si)
        pltpu.sync_copy(dst_hbm.at[pl.ds(base, EPW)], di)

        def w_desc(j, bslot):
            return pltpu.make_async_copy(
                rv.at[bslot], rad_hbm.at[pl.ds(base + j * CH, CH)],
                sems[bslot])

        def compute(j, bslot):
            for k in range(CH // LANES):
                sv = si[pl.ds(j * CH + k * LANES, LANES)] * CPAD
                dv = di[pl.ds(j * CH + k * LANES, LANES)] * CPAD
                r = jnp.zeros((LANES,), jnp.float32)
                for c in range(3):
                    xs = plsc.load_gather(ctab, [sv + c])
                    xd = plsc.load_gather(ctab, [dv + c])
                    t = xs - xd
                    r = r + t * t
                rv[bslot, pl.ds(k * LANES, LANES)] = r

        def grp(g, carry):
            for bslot in range(RING):
                j = g * RING + bslot

                @pl.when(g > 0)
                def _wait():
                    w_desc(j - RING, bslot).wait()

                compute(j, bslot)
                w_desc(j, bslot).start()
            return carry

        lax.fori_loop(0, NGRP, grp, 0)
        for bslot in range(RING):
            w_desc((NGRP - 1) * RING + bslot, bslot).wait()

    return radial_kernel(xpad, src, dst)


RING = 5
NGRP = NCH // RING
SCH = 40
SNCH = EPW // SCH
SNGRP = SNCH // RING


def _gather_call(a, b, src, dst):
    @functools.partial(
        pl.kernel,
        out_type=jax.ShapeDtypeStruct((E, D), jnp.float32),
        mesh=_sc_mesh(),
        compiler_params=_SC_PARAMS,
        scratch_types=[
            pltpu.VMEM((EPW,), jnp.int32),
            pltpu.VMEM((EPW,), jnp.int32),
            pltpu.VMEM((RING, CH, D), jnp.float32),
            pltpu.VMEM((RING, CH, D), jnp.float32),
        ] + [pltpu.SemaphoreType.DMA] * (3 * RING),
    )
    def gather_kernel(a_hbm, b_hbm, src_hbm, dst_hbm, g_hbm,
                      si, di, b1, b2, *sems):
        sg1 = sems[0:RING]
        sg2 = sems[RING:2 * RING]
        sw = sems[2 * RING:3 * RING]
        cid = lax.axis_index("c")
        sid = lax.axis_index("s")
        wid = sid * NC + cid
        base = wid * EPW
        pltpu.sync_copy(src_hbm.at[pl.ds(base, EPW)], si)
        pltpu.sync_copy(dst_hbm.at[pl.ds(base, EPW)], di)

        def g_desc(j, bslot):
            return (pltpu.make_async_copy(a_hbm.at[si.at[pl.ds(j * CH, CH)]],
                                          b1.at[bslot], sg1[bslot]),
                    pltpu.make_async_copy(b_hbm.at[di.at[pl.ds(j * CH, CH)]],
                                          b2.at[bslot], sg2[bslot]))

        def w_desc(j, bslot):
            off = base + j * CH
            return pltpu.make_async_copy(b1.at[bslot],
                                         g_hbm.at[pl.ds(off, CH)], sw[bslot])

        def add_rows(bslot):
            def row(r, carry):
                for c in range(D // LANES):
                    sl = pl.ds(c * LANES, LANES)
                    b1[bslot, r, sl] = b1[bslot, r, sl] + b2[bslot, r, sl]
                return carry
            lax.fori_loop(0, CH, row, 0)

        for bslot in range(RING):
            for d in g_desc(bslot, bslot):
                d.start()

        def grp(g, carry):
            for bslot in range(RING):
                j = g * RING + bslot
                for d in g_desc(j, bslot):
                    d.wait()
                add_rows(bslot)
                w_desc(j, bslot).start()

            @pl.when(g < NGRP - 1)
            def _prefetch():
                for bslot in range(RING):
                    j = g * RING + bslot
                    w_desc(j, bslot).wait()
                    for d in g_desc(j + RING, bslot):
                        d.start()

            return carry

        lax.fori_loop(0, NGRP, grp, 0)
        for bslot in range(RING):
            w_desc((NGRP - 1) * RING + bslot, bslot).wait()

    return gather_kernel(a, b, src, dst)


def _scatter_call(msg, dst, zeros):
    @functools.partial(
        pl.kernel,
        out_type=jax.ShapeDtypeStruct((NC, N, D), jnp.float32),
        mesh=_sc_mesh(),
        compiler_params=_SC_PARAMS,
        scratch_types=[
            pltpu.VMEM((EPW,), jnp.int32),
            pltpu.VMEM((RING, SCH, D), jnp.float32),
            pltpu.VMEM_SHARED((N, D), jnp.float32),
        ] + [pltpu.SemaphoreType.DMA] * (2 * RING),
    )
    def scatter_kernel(msg_hbm, dst_hbm, zero_hbm, out_hbm, di, buf, acc,
                       *sems):
        sl = sems[0:RING]
        ss = sems[RING:2 * RING]
        cid = lax.axis_index("c")
        sid = lax.axis_index("s")
        wid = sid * NC + cid
        base = wid * EPW

        @pl.when(sid < NS - 1)
        def _init():
            pltpu.sync_copy(zero_hbm.at[pl.ds(sid * ROWS_DUMP, ROWS_DUMP)],
                            acc.at[pl.ds(sid * ROWS_DUMP, ROWS_DUMP)])

        @pl.when(sid == NS - 1)
        def _init_last():
            rem = N - (NS - 1) * ROWS_DUMP
            pltpu.sync_copy(zero_hbm.at[pl.ds((NS - 1) * ROWS_DUMP, rem)],
                            acc.at[pl.ds((NS - 1) * ROWS_DUMP, rem)])

        pltpu.sync_copy(dst_hbm.at[pl.ds(base, EPW)], di)
        plsc.subcore_barrier()

        def l_desc(j, bslot):
            off = base + j * SCH
            return pltpu.make_async_copy(msg_hbm.at[pl.ds(off, SCH)],
                                         buf.at[bslot], sl[bslot])

        def s_desc(j, bslot):
            return pltpu.make_async_copy(
                buf.at[bslot], acc.at[di.at[pl.ds(j * SCH, SCH)]], ss[bslot])

        for bslot in range(RING):
            l_desc(bslot, bslot).start()

        def grp(g, carry):
            for bslot in range(RING):
                j = g * RING + bslot
                l_desc(j, bslot).wait()
                s_desc(j, bslot).start(add=True)

            @pl.when(g < SNGRP - 1)
            def _prefetch():
                for bslot in range(RING):
                    j = g * RING + bslot
                    s_desc(j, bslot).wait()
                    l_desc(j + RING, bslot).start()

            return carry

        lax.fori_loop(0, SNGRP, grp, 0)
        for bslot in range(RING):
            s_desc((SNGRP - 1) * RING + bslot, bslot).wait()
        plsc.subcore_barrier()

        @pl.when(sid < NS - 1)
        def _dump():
            pltpu.sync_copy(
                acc.at[pl.ds(sid * ROWS_DUMP, ROWS_DUMP)],
                out_hbm.at[cid, pl.ds(sid * ROWS_DUMP, ROWS_DUMP)])

        @pl.when(sid == NS - 1)
        def _dump_last():
            rem = N - (NS - 1) * ROWS_DUMP
            pltpu.sync_copy(
                acc.at[pl.ds((NS - 1) * ROWS_DUMP, rem)],
                out_hbm.at[cid, pl.ds((NS - 1) * ROWS_DUMP, rem)])

    return scatter_kernel(msg, dst, zeros)


def _proj_body(h_ref, wa_ref, wb_ref, ba_ref, a_ref, b_ref):
    h = h_ref[...]
    a_ref[...] = jnp.dot(h, wa_ref[...],
                         preferred_element_type=jnp.float32) + ba_ref[...]
    b_ref[...] = jnp.dot(h, wb_ref[...], preferred_element_type=jnp.float32)


def _proj_call(h, wa, wb, eb1):
    RB = 1000
    return pl.pallas_call(
        _proj_body,
        grid=(N // RB,),
        in_specs=[
            pl.BlockSpec((RB, D), lambda i: (i, 0)),
            pl.BlockSpec((D, D), lambda i: (0, 0)),
            pl.BlockSpec((D, D), lambda i: (0, 0)),
            pl.BlockSpec((1, D), lambda i: (0, 0)),
        ],
        out_specs=[
            pl.BlockSpec((RB, D), lambda i: (i, 0)),
            pl.BlockSpec((RB, D), lambda i: (i, 0)),
        ],
        out_shape=[jax.ShapeDtypeStruct((N, D), jnp.float32)] * 2,
    )(h, wa, wb, eb1.reshape(1, D))


EB = 12800
EBR = EB // 128


def _edge_body(g_ref, rad_ref, ea_ref, wr_ref, we_ref, w2_ref, b2_ref,
               msg_ref):
    eaC = lax.dot_general(ea_ref[...].astype(jnp.bfloat16),
                          we_ref[...].astype(jnp.bfloat16),
                          (((0,), (0,)), ((), ())),
                          preferred_element_type=jnp.float32)
    t3 = (g_ref[...].reshape(EBR, 128, D) + eaC.reshape(EBR, 128, D)
          + rad_ref[0][:, :, None] * wr_ref[...][None, :, :])
    m = t3 * jax.nn.sigmoid(t3)
    u = (jnp.dot(m.reshape(EB, D).astype(jnp.bfloat16),
                 w2_ref[...].astype(jnp.bfloat16),
                 preferred_element_type=jnp.float32) + b2_ref[...])
    msg_ref[...] = u * jax.nn.sigmoid(u)


def _edge_call(g, rad2, eaT, wr, we, w2, b2):
    return pl.pallas_call(
        _edge_body,
        grid=(E // EB,),
        in_specs=[
            pl.BlockSpec((EB, D), lambda i: (i, 0)),
            pl.BlockSpec((1, EBR, 128), lambda i: (i, 0, 0)),
            pl.BlockSpec((DE, EB), lambda i: (0, i)),
            pl.BlockSpec((1, D), lambda i: (0, 0)),
            pl.BlockSpec((DE, D), lambda i: (0, 0)),
            pl.BlockSpec((D, D), lambda i: (0, 0)),
            pl.BlockSpec((1, D), lambda i: (0, 0)),
        ],
        out_specs=pl.BlockSpec((EB, D), lambda i: (i, 0)),
        out_shape=jax.ShapeDtypeStruct((E, D), jnp.float32),
    )(g, rad2, eaT, wr.reshape(1, D), we, w2, b2.reshape(1, D))


def _node_body(h_ref, p_ref, w1a_ref, w1b_ref, b1_ref, w2_ref, b2_ref,
               g_ref, be_ref, o_ref):
    h = h_ref[...]
    hn = p_ref[0] + p_ref[1]
    t = (jnp.dot(h, w1a_ref[...], preferred_element_type=jnp.float32)
         + jnp.dot(hn, w1b_ref[...], preferred_element_type=jnp.float32)
         + b1_ref[...])
    u = t * jax.nn.sigmoid(t)
    v = jnp.dot(u, w2_ref[...], preferred_element_type=jnp.float32) + b2_ref[...]
    mu = jnp.mean(v, axis=1, keepdims=True)
    dlt = v - mu
    var = jnp.mean(dlt * dlt, axis=1, keepdims=True)
    vn = dlt * lax.rsqrt(var + 1e-5) * g_ref[...] + be_ref[...]
    gl = 0.5 * vn * (1.0 + lax.erf(vn / _SQRT2))
    o_ref[...] = gl + h


def _node_proj_body(h_ref, p_ref, w1a_ref, w1b_ref, b1_ref, w2_ref, b2_ref,
                    g_ref, be_ref, nwa_ref, nwb_ref, nba_ref,
                    o_ref, a_ref, b_ref):
    h = h_ref[...]
    hn = p_ref[0] + p_ref[1]
    t = (jnp.dot(h, w1a_ref[...], preferred_element_type=jnp.float32)
         + jnp.dot(hn, w1b_ref[...], preferred_element_type=jnp.float32)
         + b1_ref[...])
    u = t * jax.nn.sigmoid(t)
    v = jnp.dot(u, w2_ref[...], preferred_element_type=jnp.float32) + b2_ref[...]
    mu = jnp.mean(v, axis=1, keepdims=True)
    dlt = v - mu
    var = jnp.mean(dlt * dlt, axis=1, keepdims=True)
    vn = dlt * lax.rsqrt(var + 1e-5) * g_ref[...] + be_ref[...]
    gl = 0.5 * vn * (1.0 + lax.erf(vn / _SQRT2))
    ho = gl + h
    o_ref[...] = ho
    a_ref[...] = jnp.dot(ho, nwa_ref[...],
                         preferred_element_type=jnp.float32) + nba_ref[...]
    b_ref[...] = jnp.dot(ho, nwb_ref[...], preferred_element_type=jnp.float32)


def _node_proj_call(h, parts, w1a, w1b, b1, w2, b2, g, be, nwa, nwb, nba):
    RB = 1000
    wspec = pl.BlockSpec((D, D), lambda i: (0, 0))
    vspec = pl.BlockSpec((1, D), lambda i: (0, 0))
    rspec = pl.BlockSpec((RB, D), lambda i: (i, 0))
    return pl.pallas_call(
        _node_proj_body,
        grid=(N // RB,),
        in_specs=[
            rspec,
            pl.BlockSpec((NC, RB, D), lambda i: (0, i, 0)),
            wspec, wspec, vspec, wspec, vspec, vspec, vspec,
            wspec, wspec, vspec,
        ],
        out_specs=[rspec, rspec, rspec],
        out_shape=[jax.ShapeDtypeStruct((N, D), jnp.float32)] * 3,
    )(h, parts, w1a, w1b, b1.reshape(1, D), w2, b2.reshape(1, D),
      g.reshape(1, D), be.reshape(1, D), nwa, nwb, nba.reshape(1, D))


def _node_call(h, parts, w1a, w1b, b1, w2, b2, g, be):
    RB = 1000
    return pl.pallas_call(
        _node_body,
        grid=(N // RB,),
        in_specs=[
            pl.BlockSpec((RB, D), lambda i: (i, 0)),
            pl.BlockSpec((NC, RB, D), lambda i: (0, i, 0)),
            pl.BlockSpec((D, D), lambda i: (0, 0)),
            pl.BlockSpec((D, D), lambda i: (0, 0)),
            pl.BlockSpec((1, D), lambda i: (0, 0)),
            pl.BlockSpec((D, D), lambda i: (0, 0)),
            pl.BlockSpec((1, D), lambda i: (0, 0)),
            pl.BlockSpec((1, D), lambda i: (0, 0)),
            pl.BlockSpec((1, D), lambda i: (0, 0)),
        ],
        out_specs=pl.BlockSpec((RB, D), lambda i: (i, 0)),
        out_shape=jax.ShapeDtypeStruct((N, D), jnp.float32),
    )(h, parts, w1a, w1b, b1.reshape(1, D), w2, b2.reshape(1, D),
      g.reshape(1, D), be.reshape(1, D))


def kernel(node_features, coord_features, edge_features, edge_index, params):
    src = edge_index[0]
    dst = edge_index[1]
    xpad = jnp.pad(coord_features, ((0, 0), (0, CPAD - 3))).reshape(-1)
    rad2 = _radial_call(xpad, src, dst).reshape(E // EB, EB // 128, 128)
    eaT = edge_features.T
    zeros = jnp.zeros((N, D), jnp.float32)

    p0, p1 = params[0], params[1]

    h = node_features
    A, B = _proj_call(h, p0['eW1'][:D], p0['eW1'][D:2 * D], p0['eb1'])
    g = _gather_call(A, B, src, dst)
    msg = _edge_call(g, rad2, eaT, p0['eW1'][2 * D],
                     p0['eW1'][2 * D + 1:], p0['eW2'], p0['eb2'])
    parts = _scatter_call(msg, dst, zeros)
    h, A, B = _node_proj_call(h, parts, p0['nW1'][:D], p0['nW1'][D:],
                              p0['nb1'], p0['nW2'], p0['nb2'], p0['ln_g'],
                              p0['ln_b'], p1['eW1'][:D], p1['eW1'][D:2 * D],
                              p1['eb1'])
    g = _gather_call(A, B, src, dst)
    msg = _edge_call(g, rad2, eaT, p1['eW1'][2 * D],
                     p1['eW1'][2 * D + 1:], p1['eW2'], p1['eb2'])
    parts = _scatter_call(msg, dst, zeros)
    h = _node_call(h, parts, p1['nW1'][:D], p1['nW1'][D:], p1['nb1'],
                   p1['nW2'], p1['nb2'], p1['ln_g'], p1['ln_b'])
    return h
